# Initial kernel scaffold; baseline (speedup 1.0000x reference)
import jax
import jax.numpy as jnp
from jax.experimental import pallas as pl


def kernel(x, a, edge_index, emb, c1_W, c1_b, c2_W, c2_b, c3_W, c3_b, r1_W, r1_b, r2_W, r2_b, n1_w, n1_b, n1_a, n2_w, n2_b, n2_a, n3_w, n3_b, n3_a, fc1_W, fc1_b, fc2_W, fc2_b, fc3_W, fc3_b):
    raise NotImplementedError("write your pallas kernel here")


# speedup vs baseline: 22.9371x; 22.9371x over previous
import functools

import jax
import jax.numpy as jnp
import numpy as np
from jax import lax
from jax.experimental import pallas as pl
from jax.experimental.pallas import tpu as pltpu
from jax.experimental.pallas import tpu_sc as plsc

B = 256
BOARD = 14
PG = BOARD * BOARD
N = B * PG
E = 802816
F0, F1, F2 = 18, 36, 72
A_DIM = 196
H1, H2 = 512, 512
EPS = 1e-5

NC = 2
NS = 16
NW = NC * NS
CH = 128
EPW = E // NW
NCHUNK = EPW // CH
NPT = N // NS

@functools.cache
def _sc_mesh():
  return plsc.VectorSubcoreMesh(
      core_axis_name="c", subcore_axis_name="s", num_cores=NC, num_subcores=NS)


@functools.cache
def _make_deg():
  @functools.partial(
      pl.kernel,
      out_type=jax.ShapeDtypeStruct((NC, N, 1), jnp.float32),
      mesh=_sc_mesh(),
      compiler_params=pltpu.CompilerParams(use_tc_tiling_on_sc=False),
      scratch_types=[
          pltpu.VMEM((NCHUNK, CH), jnp.int32),
          pltpu.VMEM((CH, 1), jnp.float32),
          pltpu.VMEM_SHARED((N, 1), jnp.float32),
      ],
  )
  def deg_kernel(dstw, ones_h, zeros_h, out, dst_v, ones_v, acc):
    c = lax.axis_index("c")
    s = lax.axis_index("s")
    wid = s * NC + c
    pltpu.sync_copy(zeros_h.at[pl.ds(s * NPT, NPT)], acc.at[pl.ds(s * NPT, NPT)])
    pltpu.sync_copy(ones_h, ones_v)
    pltpu.sync_copy(dstw.at[wid], dst_v)
    plsc.subcore_barrier()

    def body(j, carry):
      pltpu.sync_copy(ones_v, acc.at[dst_v.at[j]], add=True)
      return carry

    lax.fori_loop(0, NCHUNK, body, 0)
    plsc.subcore_barrier()
    pltpu.sync_copy(acc.at[pl.ds(s * NPT, NPT)], out.at[c, pl.ds(s * NPT, NPT)])

  return deg_kernel


@functools.cache
def _make_prop(F):
  @functools.partial(
      pl.kernel,
      out_type=jax.ShapeDtypeStruct((NC, N, F), jnp.float32),
      mesh=_sc_mesh(),
      compiler_params=pltpu.CompilerParams(use_tc_tiling_on_sc=False),
      scratch_types=[
          pltpu.VMEM((NCHUNK, CH), jnp.int32),
          pltpu.VMEM((NCHUNK, CH), jnp.int32),
          pltpu.VMEM((CH, F), jnp.float32),
          pltpu.VMEM_SHARED((N, F), jnp.float32),
          pltpu.SemaphoreType.DMA,
      ],
  )
  def prop_kernel(table, srcw, dstw, zeros_h, out, src_v, dst_v, rows, acc, gsem):
    c = lax.axis_index("c")
    s = lax.axis_index("s")
    wid = s * NC + c
    pltpu.sync_copy(zeros_h.at[pl.ds(s * NPT, NPT)], acc.at[pl.ds(s * NPT, NPT)])
    pltpu.sync_copy(srcw.at[wid], src_v)
    pltpu.sync_copy(dstw.at[wid], dst_v)
    plsc.subcore_barrier()

    def body(j, carry):
      pltpu.async_copy(table.at[src_v.at[j]], rows, gsem).wait()
      pltpu.sync_copy(rows, acc.at[dst_v.at[j]], add=True)
      return carry

    lax.fori_loop(0, NCHUNK, body, 0)
    plsc.subcore_barrier()
    pltpu.sync_copy(acc.at[pl.ds(s * NPT, NPT)], out.at[c, pl.ds(s * NPT, NPT)])

  return prop_kernel


def _deg_call(dst3, ones_h, zerosN):
  return _make_deg()(dst3, ones_h, zerosN)


def _prop18(table, src3, dst3, zeros):
  return _make_prop(F0)(table, src3, dst3, zeros)


G = 16
R = G * PG
NBLK = B // G

_Pnp = np.zeros((G, R), np.float32)
for _g in range(G):
  _Pnp[_g, _g * PG:(_g + 1) * PG] = 1.0 / PG
_Penp = np.zeros((R, G), np.float32)
for _g in range(G):
  _Penp[_g * PG:(_g + 1) * PG, _g] = 1.0


def _tc0_body(xr, er, da, db, h0_o, xt0_o, dinv_o):
  xv = xr[...]
  e = er[...]
  h0 = (jnp.where(xv == -1.0, 1.0, 0.0) * e[0:1, :]
        + jnp.where(xv == 0.0, 1.0, 0.0) * e[1:2, :]
        + jnp.where(xv == 1.0, 1.0, 0.0) * e[2:3, :])
  dinv = lax.rsqrt(da[...] + db[...] + 1.0)
  h0_o[...] = h0
  xt0_o[...] = h0 * dinv
  dinv_o[...] = dinv


def _tc0_call(xcol, emb_p, dega, degb):
  return pl.pallas_call(
      _tc0_body,
      grid=(NBLK,),
      in_specs=[
          pl.BlockSpec((R, 1), lambda i: (i, 0)),
          pl.BlockSpec((8, F0), lambda i: (0, 0)),
          pl.BlockSpec((R, 1), lambda i: (i, 0)),
          pl.BlockSpec((R, 1), lambda i: (i, 0)),
      ],
      out_specs=[
          pl.BlockSpec((R, F0), lambda i: (i, 0)),
          pl.BlockSpec((R, F0), lambda i: (i, 0)),
          pl.BlockSpec((R, 1), lambda i: (i, 0)),
      ],
      out_shape=[
          jax.ShapeDtypeStruct((N, F0), jnp.float32),
          jax.ShapeDtypeStruct((N, F0), jnp.float32),
          jax.ShapeDtypeStruct((N, 1), jnp.float32),
      ],
  )(xcol, emb_p, dega, degb)


def _post_body(sa, sb, xt, dv, rin, p, pe, W, b, nw, nb, na, rW, rb,
               *outs):
  xt_outs, resn_o = outs[:-1], outs[-1]
  dinv = dv[...]
  agg = (sa[...] + sb[...] + xt[...]) * dinv
  h = agg @ W[...] + b[...]
  mean = p[...] @ h
  ctr = h - na[...] * (pe[...] @ mean)
  var = p[...] @ (ctr * ctr)
  rstd = lax.rsqrt(var + EPS)
  hn = nw[...] * (ctr * (pe[...] @ rstd)) + nb[...]
  hh = hn + rin[...]
  xtn = hh * dinv
  for i, o in enumerate(xt_outs):
    o[...] = xtn[:, i * F0:(i + 1) * F0]
  resn_o[...] = hh @ rW[...] + rb[...]


def _rows(f):
  return pl.BlockSpec((R, f), lambda i: (i, 0))


def _full(shape):
  return pl.BlockSpec(shape, lambda i: tuple(0 for _ in shape))


def _post_call(Fi, Fo, Fr, sa, sb, xt, dv, rin, p, pe, W, b, nw, nb, na, rW, rb):
  nsplit = Fo // F0
  return pl.pallas_call(
      _post_body,
      grid=(NBLK,),
      in_specs=[
          _rows(Fi), _rows(Fi), _rows(Fi), _rows(1), _rows(Fo),
          _full((G, R)), _full((R, G)),
          _full((Fi, Fo)), _full((1, Fo)), _full((1, Fo)), _full((1, Fo)),
          _full((1, Fo)), _full((Fo, Fr)), _full((1, Fr)),
      ],
      out_specs=[_rows(F0)] * nsplit + [_rows(Fr)],
      out_shape=[jax.ShapeDtypeStruct((N, F0), jnp.float32)] * nsplit
      + [jax.ShapeDtypeStruct((N, Fr), jnp.float32)],
  )(sa, sb, xt, dv, rin, p, pe, W, b, nw, nb, na, rW, rb)


def _pool_body(sa0, sa1, sb0, sb1, xta, xtb, dv, rin, p, pe, W, b, nw, nb, na,
               pooled_o):
  dinv = dv[...]
  agga = (sa0[...] + sa1[...] + xta[...]) * dinv
  aggb = (sb0[...] + sb1[...] + xtb[...]) * dinv
  W_ = W[...]
  h = agga @ W_[:F0] + aggb @ W_[F0:] + b[...]
  mean = p[...] @ h
  ctr = h - na[...] * (pe[...] @ mean)
  var = p[...] @ (ctr * ctr)
  rstd = lax.rsqrt(var + EPS)
  hn = nw[...] * (ctr * (pe[...] @ rstd)) + nb[...]
  hh = hn + rin[...]
  pooled_o[...] = p[...] @ hh


def _pool_call(Fi, Fo, sa0, sa1, sb0, sb1, xta, xtb, dv, rin, p, pe,
               W, b, nw, nb, na):
  return pl.pallas_call(
      _pool_body,
      grid=(NBLK,),
      in_specs=[
          _rows(F0), _rows(F0), _rows(F0), _rows(F0), _rows(F0), _rows(F0),
          _rows(1), _rows(Fo),
          _full((G, R)), _full((R, G)),
          _full((Fi, Fo)), _full((1, Fo)), _full((1, Fo)), _full((1, Fo)),
          _full((1, Fo)),
      ],
      out_specs=[pl.BlockSpec((G, Fo), lambda i: (i, 0))],
      out_shape=[jax.ShapeDtypeStruct((B, Fo), jnp.float32)],
  )(sa0, sa1, sb0, sb1, xta, xtb, dv, rin, p, pe, W, b, nw, nb, na)[0]


def _mlp_body(pr, ar, w1p, w1a, b1, w2, b2, w3, b3, out_o):
  z = jnp.maximum(pr[...] @ w1p[...] + ar[...] @ w1a[...] + b1[...], 0.0)
  z = jnp.maximum(z @ w2[...] + b2[...], 0.0)
  out_o[...] = z @ w3[...] + b3[...]


def _mlp_call(pooled, a, w1p, w1a, b1, w2, b2, w3, b3):
  return pl.pallas_call(
      _mlp_body,
      out_shape=jax.ShapeDtypeStruct((B, 1), jnp.float32),
  )(pooled, a, w1p, w1a, b1, w2, b2, w3, b3)


def kernel(x, a, edge_index, emb, c1_W, c1_b, c2_W, c2_b, c3_W, c3_b,
           r1_W, r1_b, r2_W, r2_b,
           n1_w, n1_b, n1_a, n2_w, n2_b, n2_a, n3_w, n3_b, n3_a,
           fc1_W, fc1_b, fc2_W, fc2_b, fc3_W, fc3_b):
  ei = edge_index.astype(jnp.int32)
  src3 = ei[0].reshape(NW, NCHUNK, CH)
  dst3 = ei[1].reshape(NW, NCHUNK, CH)
  ones_h = jnp.ones((CH, 1), jnp.float32)
  zerosN = jnp.zeros((N, 1), jnp.float32)
  zeros18 = jnp.zeros((N, F0), jnp.float32)
  p = jnp.asarray(_Pnp)
  pe = jnp.asarray(_Penp)

  degp = _deg_call(dst3, ones_h, zerosN)
  dega = degp[0]
  degb = degp[1]

  xcol = x.reshape(N, 1)
  emb_p = jnp.zeros((8, F0), jnp.float32).at[:3].set(emb)
  h0, xt0, dinv = _tc0_call(xcol, emb_p, dega, degb)

  r2 = lambda v: v.reshape(1, -1)
  S1 = _prop18(xt0, src3, dst3, zeros18)
  xt1, res1 = _post_call(F0, F0, F1, S1[0], S1[1], xt0, dinv, h0, p, pe,
                         c1_W, r2(c1_b), r2(n1_w), r2(n1_b), r2(n1_a),
                         r1_W, r2(r1_b))
  S2 = _prop18(xt1, src3, dst3, zeros18)
  xt2a, xt2b, res2 = _post_call(F0, F1, F2, S2[0], S2[1], xt1, dinv, res1,
                                p, pe, c2_W, r2(c2_b), r2(n2_w), r2(n2_b),
                                r2(n2_a), r2_W, r2(r2_b))
  S3a = _prop18(xt2a, src3, dst3, zeros18)
  S3b = _prop18(xt2b, src3, dst3, zeros18)
  pooled = _pool_call(F1, F2, S3a[0], S3a[1], S3b[0], S3b[1], xt2a, xt2b,
                      dinv, res2, p, pe,
                      c3_W, r2(c3_b), r2(n3_w), r2(n3_b), r2(n3_a))

  out = _mlp_call(pooled, a, fc1_W[:F2], fc1_W[F2:], r2(fc1_b),
                  fc2_W, r2(fc2_b), fc3_W, r2(fc3_b))
  return out


# speedup vs baseline: 25.4937x; 1.1115x over previous
import functools

import jax
import jax.numpy as jnp
import numpy as np
from jax import lax
from jax.experimental import pallas as pl
from jax.experimental.pallas import tpu as pltpu
from jax.experimental.pallas import tpu_sc as plsc

B = 256
BOARD = 14
PG = BOARD * BOARD
N = B * PG
E = 802816
F0, F1, F2 = 18, 36, 72
A_DIM = 196
H1, H2 = 512, 512
EPS = 1e-5

NC = 2
NS = 16
NW = NC * NS
CH = 128
EPW = E // NW
NCHUNK = EPW // CH
NPT = N // NS

@functools.cache
def _sc_mesh():
  return plsc.VectorSubcoreMesh(
      core_axis_name="c", subcore_axis_name="s", num_cores=NC, num_subcores=NS)


@functools.cache
def _make_deg():
  @functools.partial(
      pl.kernel,
      out_type=jax.ShapeDtypeStruct((NC, N, 1), jnp.float32),
      mesh=_sc_mesh(),
      compiler_params=pltpu.CompilerParams(use_tc_tiling_on_sc=False),
      scratch_types=[
          pltpu.VMEM((NCHUNK, CH), jnp.int32),
          pltpu.VMEM((CH, 1), jnp.float32),
          pltpu.VMEM_SHARED((N, 1), jnp.float32),
      ],
  )
  def deg_kernel(dstw, ones_h, zeros_h, out, dst_v, ones_v, acc):
    c = lax.axis_index("c")
    s = lax.axis_index("s")
    wid = s * NC + c
    pltpu.sync_copy(zeros_h.at[pl.ds(s * NPT, NPT)], acc.at[pl.ds(s * NPT, NPT)])
    pltpu.sync_copy(ones_h, ones_v)
    pltpu.sync_copy(dstw.at[wid], dst_v)
    plsc.subcore_barrier()

    def body(j, carry):
      pltpu.sync_copy(ones_v, acc.at[dst_v.at[j]], add=True)
      return carry

    lax.fori_loop(0, NCHUNK, body, 0)
    plsc.subcore_barrier()
    pltpu.sync_copy(acc.at[pl.ds(s * NPT, NPT)], out.at[c, pl.ds(s * NPT, NPT)])

  return deg_kernel


@functools.cache
def _make_prop(F):
  @functools.partial(
      pl.kernel,
      out_type=jax.ShapeDtypeStruct((NC, N, F), jnp.float32),
      mesh=_sc_mesh(),
      compiler_params=pltpu.CompilerParams(use_tc_tiling_on_sc=False),
      scratch_types=[
          pltpu.VMEM((NCHUNK, CH), jnp.int32),
          pltpu.VMEM((NCHUNK, CH), jnp.int32),
          pltpu.VMEM((CH, F), jnp.float32),
          pltpu.VMEM_SHARED((N, F), jnp.float32),
          pltpu.SemaphoreType.DMA,
      ],
  )
  def prop_kernel(table, srcw, dstw, zeros_h, out, src_v, dst_v, rows,
                  acc, gsem):
    c = lax.axis_index("c")
    s = lax.axis_index("s")
    wid = s * NC + c
    pltpu.sync_copy(zeros_h.at[pl.ds(s * NPT, NPT)], acc.at[pl.ds(s * NPT, NPT)])
    pltpu.sync_copy(srcw.at[wid], src_v)
    pltpu.sync_copy(dstw.at[wid], dst_v)
    plsc.subcore_barrier()

    def body(j, carry):
      pltpu.async_copy(table.at[src_v.at[j]], rows, gsem).wait()
      pltpu.sync_copy(rows, acc.at[dst_v.at[j]], add=True)
      return carry

    lax.fori_loop(0, NCHUNK, body, 0)
    plsc.subcore_barrier()
    pltpu.sync_copy(acc.at[pl.ds(s * NPT, NPT)], out.at[c, pl.ds(s * NPT, NPT)])

  return prop_kernel


def _deg_call(dst3, ones_h, zerosN):
  return _make_deg()(dst3, ones_h, zerosN)


def _prop18(table, src3, dst3, zeros):
  return _make_prop(F0)(table, src3, dst3, zeros)


G = 16
R = G * PG
NBLK = B // G

_Pnp = np.zeros((G, R), np.float32)
for _g in range(G):
  _Pnp[_g, _g * PG:(_g + 1) * PG] = 1.0 / PG
_Penp = np.zeros((R, G), np.float32)
for _g in range(G):
  _Penp[_g * PG:(_g + 1) * PG, _g] = 1.0


def _tc0_body(xr, er, dg, h0_o, xt0_o, dinv_o):
  xv = xr[...]
  e = er[...]
  h0 = (jnp.where(xv == -1.0, 1.0, 0.0) * e[0:1, :]
        + jnp.where(xv == 0.0, 1.0, 0.0) * e[1:2, :]
        + jnp.where(xv == 1.0, 1.0, 0.0) * e[2:3, :])
  dinv = lax.rsqrt(dg[0] + dg[1] + 1.0)
  h0_o[...] = h0
  xt0_o[...] = h0 * dinv
  dinv_o[...] = dinv


def _tc0_call(xcol, emb_p, degp):
  return pl.pallas_call(
      _tc0_body,
      grid=(NBLK,),
      in_specs=[
          pl.BlockSpec((R, 1), lambda i: (i, 0)),
          pl.BlockSpec((8, F0), lambda i: (0, 0)),
          pl.BlockSpec((NC, R, 1), lambda i: (0, i, 0)),
      ],
      out_specs=[
          pl.BlockSpec((R, F0), lambda i: (i, 0)),
          pl.BlockSpec((R, F0), lambda i: (i, 0)),
          pl.BlockSpec((R, 1), lambda i: (i, 0)),
      ],
      out_shape=[
          jax.ShapeDtypeStruct((N, F0), jnp.float32),
          jax.ShapeDtypeStruct((N, F0), jnp.float32),
          jax.ShapeDtypeStruct((N, 1), jnp.float32),
      ],
  )(xcol, emb_p, degp)


def _post_body(s3, xt, dv, rin, p, pe, W, b, nw, nb, na, rW, rb,
               *outs):
  xt_outs, resn_o = outs[:-1], outs[-1]
  dinv = dv[...]
  agg = (s3[0] + s3[1] + xt[...]) * dinv
  h = agg @ W[...] + b[...]
  mean = p[...] @ h
  ctr = h - na[...] * (pe[...] @ mean)
  var = p[...] @ (ctr * ctr)
  rstd = lax.rsqrt(var + EPS)
  hn = nw[...] * (ctr * (pe[...] @ rstd)) + nb[...]
  hh = hn + rin[...]
  xtn = hh * dinv
  for i, o in enumerate(xt_outs):
    o[...] = xtn[:, i * F0:(i + 1) * F0]
  resn_o[...] = hh @ rW[...] + rb[...]


def _rows(f):
  return pl.BlockSpec((R, f), lambda i: (i, 0))


def _full(shape):
  return pl.BlockSpec(shape, lambda i: tuple(0 for _ in shape))


def _post_call(Fi, Fo, Fr, s3, xt, dv, rin, p, pe, W, b, nw, nb, na, rW, rb):
  nsplit = Fo // F0
  return pl.pallas_call(
      _post_body,
      grid=(NBLK,),
      in_specs=[
          pl.BlockSpec((NC, R, Fi), lambda i: (0, i, 0)),
          _rows(Fi), _rows(1), _rows(Fo),
          _full((G, R)), _full((R, G)),
          _full((Fi, Fo)), _full((1, Fo)), _full((1, Fo)), _full((1, Fo)),
          _full((1, Fo)), _full((Fo, Fr)), _full((1, Fr)),
      ],
      out_specs=[_rows(F0)] * nsplit + [_rows(Fr)],
      out_shape=[jax.ShapeDtypeStruct((N, F0), jnp.float32)] * nsplit
      + [jax.ShapeDtypeStruct((N, Fr), jnp.float32)],
  )(s3, xt, dv, rin, p, pe, W, b, nw, nb, na, rW, rb)


def _pool_body(s3a, s3b, xta, xtb, dv, rin, p, pe, W, b, nw, nb, na,
               pooled_o):
  dinv = dv[...]
  agga = (s3a[0] + s3a[1] + xta[...]) * dinv
  aggb = (s3b[0] + s3b[1] + xtb[...]) * dinv
  W_ = W[...]
  h = agga @ W_[:F0] + aggb @ W_[F0:] + b[...]
  mean = p[...] @ h
  ctr = h - na[...] * (pe[...] @ mean)
  var = p[...] @ (ctr * ctr)
  rstd = lax.rsqrt(var + EPS)
  hn = nw[...] * (ctr * (pe[...] @ rstd)) + nb[...]
  hh = hn + rin[...]
  pooled_o[...] = p[...] @ hh


def _pool_call(Fi, Fo, s3a, s3b, xta, xtb, dv, rin, p, pe, W, b, nw, nb, na):
  return pl.pallas_call(
      _pool_body,
      grid=(NBLK,),
      in_specs=[
          pl.BlockSpec((NC, R, F0), lambda i: (0, i, 0)),
          pl.BlockSpec((NC, R, F0), lambda i: (0, i, 0)),
          _rows(F0), _rows(F0),
          _rows(1), _rows(Fo),
          _full((G, R)), _full((R, G)),
          _full((Fi, Fo)), _full((1, Fo)), _full((1, Fo)), _full((1, Fo)),
          _full((1, Fo)),
      ],
      out_specs=[pl.BlockSpec((G, Fo), lambda i: (i, 0))],
      out_shape=[jax.ShapeDtypeStruct((B, Fo), jnp.float32)],
  )(s3a, s3b, xta, xtb, dv, rin, p, pe, W, b, nw, nb, na)[0]


def _mlp_body(pr, ar, w1p, w1a, b1, w2, b2, w3, b3, out_o):
  z = jnp.maximum(pr[...] @ w1p[...] + ar[...] @ w1a[...] + b1[...], 0.0)
  z = jnp.maximum(z @ w2[...] + b2[...], 0.0)
  out_o[...] = z @ w3[...] + b3[...]


def _mlp_call(pooled, a, w1p, w1a, b1, w2, b2, w3, b3):
  return pl.pallas_call(
      _mlp_body,
      out_shape=jax.ShapeDtypeStruct((B, 1), jnp.float32),
  )(pooled, a, w1p, w1a, b1, w2, b2, w3, b3)


def kernel(x, a, edge_index, emb, c1_W, c1_b, c2_W, c2_b, c3_W, c3_b,
           r1_W, r1_b, r2_W, r2_b,
           n1_w, n1_b, n1_a, n2_w, n2_b, n2_a, n3_w, n3_b, n3_a,
           fc1_W, fc1_b, fc2_W, fc2_b, fc3_W, fc3_b):
  ei = edge_index.astype(jnp.int32)
  src3 = ei[0].reshape(NW, NCHUNK, CH)
  dst3 = ei[1].reshape(NW, NCHUNK, CH)
  ones_h = jnp.ones((CH, 1), jnp.float32)
  zerosN = jnp.zeros((N, 1), jnp.float32)
  zeros18 = jnp.zeros((N, F0), jnp.float32)
  p = jnp.asarray(_Pnp)
  pe = jnp.asarray(_Penp)

  degp = _deg_call(dst3, ones_h, zerosN)

  xcol = x.reshape(N, 1)
  emb_p = jnp.zeros((8, F0), jnp.float32).at[:3].set(emb)
  h0, xt0, dinv = _tc0_call(xcol, emb_p, degp)

  r2 = lambda v: v.reshape(1, -1)
  S1 = _prop18(xt0, src3, dst3, zeros18)
  xt1, res1 = _post_call(F0, F0, F1, S1, xt0, dinv, h0, p, pe,
                         c1_W, r2(c1_b), r2(n1_w), r2(n1_b), r2(n1_a),
                         r1_W, r2(r1_b))
  S2 = _prop18(xt1, src3, dst3, zeros18)
  xt2a, xt2b, res2 = _post_call(F0, F1, F2, S2, xt1, dinv, res1,
                                p, pe, c2_W, r2(c2_b), r2(n2_w), r2(n2_b),
                                r2(n2_a), r2_W, r2(r2_b))
  S3a = _prop18(xt2a, src3, dst3, zeros18)
  S3b = _prop18(xt2b, src3, dst3, zeros18)
  pooled = _pool_call(F1, F2, S3a, S3b, xt2a, xt2b,
                      dinv, res2, p, pe,
                      c3_W, r2(c3_b), r2(n3_w), r2(n3_b), r2(n3_a))

  out = _mlp_call(pooled, a, fc1_W[:F2], fc1_W[F2:], r2(fc1_b),
                  fc2_W, r2(fc2_b), fc3_W, r2(fc3_b))
  return out
